# baseline (speedup 1.0000x reference)
import functools

import jax
import jax.numpy as jnp
from jax import lax
from jax.experimental import pallas as pl
from jax.experimental.pallas import tpu as pltpu
from jax.experimental.pallas import tpu_sc as plsc

_C, _T, _H, _W = 3, 64, 512, 512
_ALPHA = 4
_TS = _T // _ALPHA
_NW = 32


def _copy_body(x_ref, o_ref):
    o_ref[...] = x_ref[...]


_TC_NPIECE = 12
_TC_FPP = _T // (_TC_NPIECE // _C)


def _tc_copy_body(x_hbm, o_hbm, *sems):
    cps = []
    for j in range(_TC_NPIECE):
        c = j // (_TC_NPIECE // _C)
        lo = (j % (_TC_NPIECE // _C)) * _TC_FPP
        cps.append(
            pltpu.make_async_copy(
                x_hbm.at[c, pl.ds(lo, _TC_FPP)],
                o_hbm.at[c, pl.ds(lo, _TC_FPP)],
                sems[j],
            )
        )
    for cp in cps:
        cp.start()
    for cp in cps:
        cp.wait()


_tc_fast_copy = pl.pallas_call(
    _tc_copy_body,
    in_specs=[pl.BlockSpec(memory_space=pltpu.MemorySpace.HBM)],
    out_specs=pl.BlockSpec(memory_space=pltpu.MemorySpace.HBM),
    out_shape=jax.ShapeDtypeStruct((_C, _T, _H, _W), jnp.float32),
    scratch_shapes=[pltpu.SemaphoreType.DMA] * _TC_NPIECE,
)


def _make_sc_ring_copy(out_shape, nbuf, crows, chunks_total, src_at, dst_at):
    nch = chunks_total // _NW
    nsuper = nch // nbuf

    @functools.partial(
        pl.kernel,
        mesh=plsc.VectorSubcoreMesh(core_axis_name="c", subcore_axis_name="s"),
        out_type=jax.ShapeDtypeStruct(out_shape, jnp.float32),
        scratch_types=[
            [pltpu.VMEM((crows, _W), jnp.float32)] * nbuf,
            [pltpu.SemaphoreType.DMA] * nbuf,
            [pltpu.SemaphoreType.DMA] * nbuf,
        ],
    )
    def sc_copy(in_hbm, out_hbm, bufs, rsems, wsems):
        wid = lax.axis_index("s") * 2 + lax.axis_index("c")
        base = wid * nch

        for j in range(nbuf):
            pltpu.async_copy(src_at(in_hbm, base + j), bufs[j], rsems[j])

        def body(it, carry):
            g0 = base + it * nbuf
            for j in range(nbuf):
                pltpu.make_async_copy(
                    src_at(in_hbm, g0 + j), bufs[j], rsems[j]
                ).wait()
                pltpu.async_copy(bufs[j], dst_at(out_hbm, g0 + j), wsems[j])

            @pl.when(it < nsuper - 1)
            def _():
                for j in range(nbuf):
                    pltpu.make_async_copy(
                        bufs[j], dst_at(out_hbm, g0 + j), wsems[j]
                    ).wait()
                    pltpu.async_copy(
                        src_at(in_hbm, g0 + nbuf + j), bufs[j], rsems[j]
                    )

            return carry

        lax.fori_loop(0, nsuper, body, 0)

        g_last = base + (nsuper - 1) * nbuf
        for j in range(nbuf):
            pltpu.make_async_copy(
                bufs[j], dst_at(out_hbm, g_last + j), wsems[j]
            ).wait()

    return sc_copy


_G_CROWS = 32
_G_CPF = _H // _G_CROWS
_G_TOTAL = _C * _TS * _G_CPF


def _gather_src(ref, g):
    j = g // _G_CPF
    r = (g % _G_CPF) * _G_CROWS
    c = j // _TS
    t = (21 * (j % _TS)) // 5
    return ref.at[c, t, pl.ds(r, _G_CROWS), :]


def _gather_dst(ref, g):
    j = g // _G_CPF
    r = (g % _G_CPF) * _G_CROWS
    return ref.at[j // _TS, j % _TS, pl.ds(r, _G_CROWS), :]


_sc_slow_gather = _make_sc_ring_copy(
    (_C, _TS, _H, _W), 4, _G_CROWS, _G_TOTAL, _gather_src, _gather_dst
)


def kernel(frames):
    slow = _sc_slow_gather(frames)
    fast = _tc_fast_copy(frames)
    return (slow, fast)


# speedup vs baseline: 35.2468x; 35.2468x over previous
import functools

import jax
import jax.numpy as jnp
from jax import lax
from jax.experimental import pallas as pl
from jax.experimental.pallas import tpu as pltpu
from jax.experimental.pallas import tpu_sc as plsc

_C, _T, _H, _W = 3, 64, 512, 512
_ALPHA = 4
_TS = _T // _ALPHA
_NW = 32


def _copy_body(x_ref, o_ref):
    o_ref[...] = x_ref[...]


_tc_fast_copy = pl.pallas_call(
    _copy_body,
    grid=(_T // 4,),
    in_specs=[pl.BlockSpec((_C, 4, _H, _W), lambda i: (0, i, 0, 0))],
    out_specs=pl.BlockSpec((_C, 4, _H, _W), lambda i: (0, i, 0, 0)),
    out_shape=jax.ShapeDtypeStruct((_C, _T, _H, _W), jnp.float32),
)


def _make_sc_ring_copy(out_shape, nbuf, crows, chunks_total, src_at, dst_at):
    nch = chunks_total // _NW
    nsuper = nch // nbuf

    @functools.partial(
        pl.kernel,
        mesh=plsc.VectorSubcoreMesh(core_axis_name="c", subcore_axis_name="s"),
        out_type=jax.ShapeDtypeStruct(out_shape, jnp.float32),
        scratch_types=[
            [pltpu.VMEM((crows, _W), jnp.float32)] * nbuf,
            [pltpu.SemaphoreType.DMA] * nbuf,
            [pltpu.SemaphoreType.DMA] * nbuf,
        ],
    )
    def sc_copy(in_hbm, out_hbm, bufs, rsems, wsems):
        wid = lax.axis_index("s") * 2 + lax.axis_index("c")
        base = wid * nch

        for j in range(nbuf):
            pltpu.async_copy(src_at(in_hbm, base + j), bufs[j], rsems[j])

        def body(it, carry):
            g0 = base + it * nbuf
            for j in range(nbuf):
                pltpu.make_async_copy(
                    src_at(in_hbm, g0 + j), bufs[j], rsems[j]
                ).wait()
                pltpu.async_copy(bufs[j], dst_at(out_hbm, g0 + j), wsems[j])

            @pl.when(it < nsuper - 1)
            def _():
                for j in range(nbuf):
                    pltpu.make_async_copy(
                        bufs[j], dst_at(out_hbm, g0 + j), wsems[j]
                    ).wait()
                    pltpu.async_copy(
                        src_at(in_hbm, g0 + nbuf + j), bufs[j], rsems[j]
                    )

            return carry

        lax.fori_loop(0, nsuper, body, 0)

        g_last = base + (nsuper - 1) * nbuf
        for j in range(nbuf):
            pltpu.make_async_copy(
                bufs[j], dst_at(out_hbm, g_last + j), wsems[j]
            ).wait()

    return sc_copy


_G_CROWS = 32
_G_CPF = _H // _G_CROWS
_G_TOTAL = _C * _TS * _G_CPF


def _gather_src(ref, g):
    j = g // _G_CPF
    r = (g % _G_CPF) * _G_CROWS
    c = j // _TS
    t = (21 * (j % _TS)) // 5
    return ref.at[c, t, pl.ds(r, _G_CROWS), :]


def _gather_dst(ref, g):
    j = g // _G_CPF
    r = (g % _G_CPF) * _G_CROWS
    return ref.at[j // _TS, j % _TS, pl.ds(r, _G_CROWS), :]


_sc_slow_gather = _make_sc_ring_copy(
    (_C, _TS, _H, _W), 4, _G_CROWS, _G_TOTAL, _gather_src, _gather_dst
)


def kernel(frames):
    slow = _sc_slow_gather(frames)
    fast = _tc_fast_copy(frames)
    return (slow, fast)
